# baseline (speedup 1.0000x reference)
import jax
import jax.numpy as jnp
from jax.experimental import pallas as pl
from jax.experimental.pallas import tpu as pltpu

_B, _T, _P, _G, _C = 4, 8, 16384, 32, 5
_BT = _B * _T
_R = 128
_F = _T


def _smooth_l1(x):
    ax = jnp.abs(x)
    return jnp.where(ax < 1.0, 0.5 * x * x, ax - 0.5)


def _red(op, x):
    return op(op(x, axis=2, keepdims=True), axis=1, keepdims=True)


def _cumsum_rowmajor(x):
    c = x
    s = 1
    while s < _R:
        c = c + jnp.concatenate(
            [jnp.zeros((_F, _R, s), jnp.int32), c[:, :, : _R - s]], axis=2)
        s *= 2
    row_tot = c[:, :, _R - 1 : _R]
    r = row_tot
    s = 1
    while s < _R:
        r = r + jnp.concatenate(
            [jnp.zeros((_F, s, 1), jnp.int32), r[:, : _R - s]], axis=1)
        s *= 2
    return c + (r - row_tot)


def _frame_kernel(tpow_ref, tgt_ref, prm_ref, pm_ref, conf_ref, pri_ref,
                  ll_ref, lc_ref):
    tp = tpow_ref[0]
    t1 = tp[1].reshape(_F, 1, 1)
    t2 = tp[2].reshape(_F, 1, 1)

    prm = prm_ref[0]
    loc = [prm[c, 0][None] + prm[c, 1][None] * t1 + prm[c, 2][None] * t2
           for c in range(4)]

    pri = pri_ref[...]
    pax = pri[0] - pri[2] * 0.5
    pay = pri[1] - pri[3] * 0.5
    pbx = pri[0] + pri[2] * 0.5
    pby = pri[1] + pri[3] * 0.5
    area_p = (pbx - pax) * (pby - pay)

    lin = (jax.lax.broadcasted_iota(jnp.int32, (_R, _R), 0) * _R
           + jax.lax.broadcasted_iota(jnp.int32, (_R, _R), 1))

    def attr(j, g):
        return tgt_ref[0, j, g].reshape(_F, 1, 1)

    def iou_body(g, carry):
        bto, bti = carry
        cx, cy = attr(0, g), attr(1, g)
        w, h = attr(2, g), attr(3, g)
        vld = attr(5, g)
        ax = cx - w * 0.5
        ay = cy - h * 0.5
        bx = cx + w * 0.5
        by = cy + h * 0.5
        area_t = (bx - ax) * (by - ay)
        wx = jnp.clip(jnp.minimum(pbx[None], bx) - jnp.maximum(pax[None], ax),
                      0.0)
        wy = jnp.clip(jnp.minimum(pby[None], by) - jnp.maximum(pay[None], ay),
                      0.0)
        inter = wx * wy
        ovg = inter / (area_p[None] + area_t - inter + 1e-9)
        ovg = jnp.where(vld > 0.5, ovg, -1.0)
        better = ovg > bto
        bto = jnp.where(better, ovg, bto)
        bti = jnp.where(better, g, bti)
        m_g = _red(jnp.max, ovg)
        first_g = _red(jnp.min, jnp.where(ovg == m_g, lin[None], _P))
        hit = (lin[None] == first_g) & (m_g > -0.5)
        bto = jnp.where(hit, 2.0, bto)
        bti = jnp.where(hit, g, bti)
        return bto, bti

    bto0 = jnp.full((_F, _R, _R), -2.0, jnp.float32)
    bti0 = jnp.zeros((_F, _R, _R), jnp.int32)
    bto, bti = jax.lax.fori_loop(0, _G, iou_body, (bto0, bti0))

    def gather_body(g, carry):
        oh = bti == g
        out = []
        for j, m in enumerate(carry):
            out.append(jnp.where(oh, attr(j, g), m))
        return tuple(out)

    zero = jnp.zeros((_F, _R, _R), jnp.float32)
    mcx, mcy, mw, mh, mlab = jax.lax.fori_loop(
        0, _G, gather_body, (zero, zero, zero, zero, zero))

    pos = bto >= 0.5
    num_pos = _red(jnp.sum, pos.astype(jnp.int32))
    denom = jnp.maximum(num_pos, 1).astype(jnp.float32)

    matched = [mcx, mcy, mw, mh]
    sl = jnp.where(pos, _smooth_l1(loc[0] - matched[0]), 0.0)
    for c in range(1, 4):
        sl = sl + jnp.where(pos, _smooth_l1(loc[c] - matched[c]), 0.0)
    loss_l = _red(jnp.sum, sl) / denom

    conf = conf_ref[0]
    cmax = conf[:, 0]
    for c in range(1, _C):
        cmax = jnp.maximum(cmax, conf[:, c])
    sumexp = jnp.exp(conf[:, 0] - cmax)
    for c in range(1, _C):
        sumexp = sumexp + jnp.exp(conf[:, c] - cmax)
    lse = cmax + jnp.log(sumexp)
    conf_t = jnp.where(pos, mlab.astype(jnp.int32), 0)
    conf_sel = conf[:, 0]
    for c in range(1, _C):
        conf_sel = jnp.where(conf_t == c, conf[:, c], conf_sel)
    ce = lse - conf_sel

    negm = jnp.logical_not(pos)
    k = jnp.minimum(3 * num_pos, _P - num_pos)
    bits = jax.lax.bitcast_convert_type(ce, jnp.int32)
    cand = jnp.where(negm, bits, -1)

    def bs_body(_, carry):
        lo, hi = carry
        mid = lo + (hi - lo + 1) // 2
        cnt = _red(jnp.sum, (cand >= mid).astype(jnp.int32))
        ge = cnt >= k
        return (jnp.where(ge, mid, lo), jnp.where(ge, hi, mid - 1))

    pref, _ = jax.lax.fori_loop(
        0, 31, bs_body,
        (jnp.zeros((_F, 1, 1), jnp.int32),
         jnp.full((_F, 1, 1), 0x7F800000, jnp.int32)))

    gt = cand > pref
    cnt_gt = _red(jnp.sum, gt.astype(jnp.int32))
    need = k - cnt_gt
    ties = cand == pref
    prefix = _cumsum_rowmajor(ties.astype(jnp.int32))
    neg_sel = gt | (ties & (prefix <= need))

    sel = pos | neg_sel
    loss_c = _red(jnp.sum, jnp.where(sel, ce, 0.0)) / denom

    pm = pm_ref[0]
    y = pos.astype(jnp.float32)
    bce = (jnp.maximum(pm, 0.0) - pm * y
           + jnp.log1p(jnp.exp(-jnp.abs(pm))))
    loss_pm = _red(jnp.sum, jnp.where(sel, bce, 0.0)) / denom

    ll_ref[0] = loss_l
    lc_ref[0] = loss_c + loss_pm


def _run(tpow, tgt, prm, pm, conf, pri):
    return pl.pallas_call(
        _frame_kernel,
        grid=(1,),
        in_specs=[
            pl.BlockSpec((1, 3, _T, 1), lambda i: (0, 0, 0, 0)),
            pl.BlockSpec((1, 6, _G, _T, 1), lambda i: (0, 0, 0, 0, 0)),
            pl.BlockSpec((1, 4, 3, _R, _R), lambda i: (0, 0, 0, 0, 0)),
            pl.BlockSpec((1, _T, _R, _R), lambda i: (0, 0, 0, 0)),
            pl.BlockSpec((1, _T, _C, _R, _R), lambda i: (0, 0, 0, 0, 0)),
            pl.BlockSpec((4, _R, _R), lambda i: (0, 0, 0)),
        ],
        out_specs=[
            pl.BlockSpec((1, _T, 1, 1), lambda i: (0, 0, 0, 0)),
            pl.BlockSpec((1, _T, 1, 1), lambda i: (0, 0, 0, 0)),
        ],
        out_shape=[
            jax.ShapeDtypeStruct((1, _T, 1, 1), jnp.float32),
            jax.ShapeDtypeStruct((1, _T, 1, 1), jnp.float32),
        ],
    )(tpow, tgt, prm, pm, conf, pri)


def kernel(parameters, p_m_datas, p_c_datas, priors, targets, times):
    tpow = jnp.stack(
        [jnp.ones_like(times), times, times * times], axis=1
    ).reshape(_B, 3, _T, 1)
    pri = priors.transpose(1, 0).reshape(4, _R, _R)
    lls, lcs = [], []
    for b in range(_B):
        prm = parameters[b].transpose(1, 2, 0).reshape(1, 4, 3, _R, _R)
        pm = p_m_datas[b].reshape(1, _T, _R, _R)
        conf = p_c_datas[b].transpose(0, 2, 1).reshape(1, _T, _C, _R, _R)
        tgt = targets[b].transpose(2, 1, 0).reshape(1, 6, _G, _T, 1)
        ll_b, lc_b = _run(tpow[b : b + 1], tgt, prm, pm, conf, pri)
        lls.append(ll_b)
        lcs.append(lc_b)
    ll = jnp.concatenate(lls, axis=0)
    lc = jnp.concatenate(lcs, axis=0)
    return (ll.reshape(_B, _T).sum(axis=0), lc.reshape(_B, _T).sum(axis=0))


# speedup vs baseline: 1.0893x; 1.0893x over previous
import jax
import jax.numpy as jnp
from jax.experimental import pallas as pl
from jax.experimental.pallas import tpu as pltpu

_B, _T, _P, _G, _C = 4, 8, 16384, 32, 5
_BT = _B * _T
_R = 128
_F = _T


def _smooth_l1(x):
    ax = jnp.abs(x)
    return jnp.where(ax < 1.0, 0.5 * x * x, ax - 0.5)


def _red(op, x):
    return op(op(x, axis=2, keepdims=True), axis=1, keepdims=True)


def _cumsum_rowmajor(x):
    c = x
    s = 1
    while s < _R:
        c = c + jnp.concatenate(
            [jnp.zeros((_F, _R, s), jnp.int32), c[:, :, : _R - s]], axis=2)
        s *= 2
    row_tot = c[:, :, _R - 1 : _R]
    r = row_tot
    s = 1
    while s < _R:
        r = r + jnp.concatenate(
            [jnp.zeros((_F, s, 1), jnp.int32), r[:, : _R - s]], axis=1)
        s *= 2
    return c + (r - row_tot)


def _frame_kernel(tpow_ref, tgt_ref, prm_ref, pm_ref, conf_ref, pri_ref,
                  ll_ref, lc_ref):
    tp = tpow_ref[0]
    t1 = tp[1].reshape(_F, 1, 1)
    t2 = tp[2].reshape(_F, 1, 1)

    prm = prm_ref[0]
    loc = [prm[c, 0][None] + prm[c, 1][None] * t1 + prm[c, 2][None] * t2
           for c in range(4)]

    pri = pri_ref[...]
    pax = pri[0] - pri[2] * 0.5
    pay = pri[1] - pri[3] * 0.5
    pbx = pri[0] + pri[2] * 0.5
    pby = pri[1] + pri[3] * 0.5
    area_p = (pbx - pax) * (pby - pay)

    lin = (jax.lax.broadcasted_iota(jnp.int32, (_R, _R), 0) * _R
           + jax.lax.broadcasted_iota(jnp.int32, (_R, _R), 1))

    def attr(j, g):
        return tgt_ref[0, j, g].reshape(_F, 1, 1)

    def iou_body(g, carry):
        bto, bti = carry
        cx, cy = attr(0, g), attr(1, g)
        w, h = attr(2, g), attr(3, g)
        vld = attr(5, g)
        ax = cx - w * 0.5
        ay = cy - h * 0.5
        bx = cx + w * 0.5
        by = cy + h * 0.5
        area_t = (bx - ax) * (by - ay)
        wx = jnp.clip(jnp.minimum(pbx[None], bx) - jnp.maximum(pax[None], ax),
                      0.0)
        wy = jnp.clip(jnp.minimum(pby[None], by) - jnp.maximum(pay[None], ay),
                      0.0)
        inter = wx * wy
        ovg = inter / (area_p[None] + area_t - inter + 1e-9)
        ovg = jnp.where(vld > 0.5, ovg, -1.0)
        better = ovg > bto
        bto = jnp.where(better, ovg, bto)
        bti = jnp.where(better, g, bti)
        m_g = _red(jnp.max, ovg)
        first_g = _red(jnp.min, jnp.where(ovg == m_g, lin[None], _P))
        hit = (lin[None] == first_g) & (m_g > -0.5)
        bto = jnp.where(hit, 2.0, bto)
        bti = jnp.where(hit, g, bti)
        return bto, bti

    bto0 = jnp.full((_F, _R, _R), -2.0, jnp.float32)
    bti0 = jnp.zeros((_F, _R, _R), jnp.int32)
    bto, bti = jax.lax.fori_loop(0, _G, iou_body, (bto0, bti0))

    def gather_body(g, carry):
        oh = bti == g
        out = []
        for j, m in enumerate(carry):
            out.append(jnp.where(oh, attr(j, g), m))
        return tuple(out)

    zero = jnp.zeros((_F, _R, _R), jnp.float32)
    mcx, mcy, mw, mh, mlab = jax.lax.fori_loop(
        0, _G, gather_body, (zero, zero, zero, zero, zero))

    pos = bto >= 0.5
    num_pos = _red(jnp.sum, pos.astype(jnp.int32))
    denom = jnp.maximum(num_pos, 1).astype(jnp.float32)

    matched = [mcx, mcy, mw, mh]
    sl = jnp.where(pos, _smooth_l1(loc[0] - matched[0]), 0.0)
    for c in range(1, 4):
        sl = sl + jnp.where(pos, _smooth_l1(loc[c] - matched[c]), 0.0)
    loss_l = _red(jnp.sum, sl) / denom

    conf = conf_ref[0]
    cmax = conf[:, 0]
    for c in range(1, _C):
        cmax = jnp.maximum(cmax, conf[:, c])
    sumexp = jnp.exp(conf[:, 0] - cmax)
    for c in range(1, _C):
        sumexp = sumexp + jnp.exp(conf[:, c] - cmax)
    lse = cmax + jnp.log(sumexp)
    conf_t = jnp.where(pos, mlab.astype(jnp.int32), 0)
    conf_sel = conf[:, 0]
    for c in range(1, _C):
        conf_sel = jnp.where(conf_t == c, conf[:, c], conf_sel)
    ce = lse - conf_sel

    negm = jnp.logical_not(pos)
    k = jnp.minimum(3 * num_pos, _P - num_pos)
    bits = jax.lax.bitcast_convert_type(ce, jnp.int32)
    cand = jnp.where(negm, bits, -1)

    def bs_body(_, carry):
        lo, hi = carry
        mid = lo + (hi - lo + 1) // 2
        cnt = _red(jnp.sum, (cand >= mid).astype(jnp.int32))
        ge = cnt >= k
        return (jnp.where(ge, mid, lo), jnp.where(ge, hi, mid - 1))

    pref, _ = jax.lax.fori_loop(
        0, 31, bs_body,
        (jnp.zeros((_F, 1, 1), jnp.int32),
         jnp.full((_F, 1, 1), 0x7F800000, jnp.int32)))

    gt = cand > pref
    cnt_gt = _red(jnp.sum, gt.astype(jnp.int32))
    need = k - cnt_gt
    ties = cand == pref
    prefix = _cumsum_rowmajor(ties.astype(jnp.int32))
    neg_sel = gt | (ties & (prefix <= need))

    sel = pos | neg_sel
    loss_c = _red(jnp.sum, jnp.where(sel, ce, 0.0)) / denom

    pm = pm_ref[0]
    y = pos.astype(jnp.float32)
    bce = (jnp.maximum(pm, 0.0) - pm * y
           + jnp.log1p(jnp.exp(-jnp.abs(pm))))
    loss_pm = _red(jnp.sum, jnp.where(sel, bce, 0.0)) / denom

    ll_ref[0] = loss_l
    lc_ref[0] = loss_c + loss_pm


def _run(tpow, tgt, prm, pm, conf, pri):
    return pl.pallas_call(
        _frame_kernel,
        grid=(_B,),
        in_specs=[
            pl.BlockSpec((1, 3, _T, 1), lambda i: (i, 0, 0, 0)),
            pl.BlockSpec((1, 6, _G, _T, 1), lambda i: (i, 0, 0, 0, 0)),
            pl.BlockSpec((1, 4, 3, _R, _R), lambda i: (i, 0, 0, 0, 0)),
            pl.BlockSpec((1, _T, _R, _R), lambda i: (i, 0, 0, 0)),
            pl.BlockSpec((1, _T, _C, _R, _R), lambda i: (i, 0, 0, 0, 0)),
            pl.BlockSpec((4, _R, _R), lambda i: (0, 0, 0)),
        ],
        out_specs=[
            pl.BlockSpec((1, _T, 1, 1), lambda i: (i, 0, 0, 0)),
            pl.BlockSpec((1, _T, 1, 1), lambda i: (i, 0, 0, 0)),
        ],
        out_shape=[
            jax.ShapeDtypeStruct((_B, _T, 1, 1), jnp.float32),
            jax.ShapeDtypeStruct((_B, _T, 1, 1), jnp.float32),
        ],
    )(tpow, tgt, prm, pm, conf, pri)


def kernel(parameters, p_m_datas, p_c_datas, priors, targets, times):
    tpow = jnp.stack(
        [jnp.ones_like(times), times, times * times], axis=1
    ).reshape(_B, 3, _T, 1)
    prm = parameters.transpose(0, 2, 3, 1).reshape(_B, 4, 3, _R, _R)
    pm = p_m_datas.reshape(_B, _T, _R, _R)
    conf = p_c_datas.transpose(0, 1, 3, 2).reshape(_B, _T, _C, _R, _R)
    pri = priors.transpose(1, 0).reshape(4, _R, _R)
    tgt = targets.transpose(0, 3, 2, 1).reshape(_B, 6, _G, _T, 1)
    ll, lc = _run(tpow, tgt, prm, pm, conf, pri)
    return (ll.reshape(_B, _T).sum(axis=0), lc.reshape(_B, _T).sum(axis=0))
